# Initial kernel scaffold; baseline (speedup 1.0000x reference)
import jax
import jax.numpy as jnp
from jax.experimental import pallas as pl


def kernel(rois, probs, deltas, window):
    raise NotImplementedError("write your pallas kernel here")


# speedup vs baseline: 3.8224x; 3.8224x over previous
import functools

import jax
import jax.numpy as jnp
from jax.experimental import pallas as pl

N = 20000
C = 81
TILE = 400
NPAD = 20480
ROWS = 160
LANES = 128
MAXDET = 100
STD = (0.1, 0.1, 0.2, 0.2)
MIN_CONF = 0.7
NMS_THRESH = 0.3


def _stage_a_body(rois_ref, probs_ref, deltas_ref, win_ref,
                  y1_ref, x1_ref, y2_ref, x2_ref, cf_ref, sc_ref):
    p = probs_ref[...]
    m = jnp.max(p, axis=1, keepdims=True)
    li = jax.lax.broadcasted_iota(jnp.int32, (TILE, C), 1)
    idx = jnp.min(jnp.where(p == m, li, C), axis=1, keepdims=True)

    dl = deltas_ref[...]
    l4 = jax.lax.broadcasted_iota(jnp.int32, (TILE, 4 * C), 1)
    grp = l4 // 4
    mod = l4 - 4 * grp
    sel = grp == idx
    d = []
    for j in range(4):
        dj = jnp.sum(jnp.where(sel & (mod == j), dl, 0.0), axis=1,
                     keepdims=True)
        d.append(dj * STD[j])

    r = rois_ref[...]
    ry1 = r[:, 0:1]
    rx1 = r[:, 1:2]
    ry2 = r[:, 2:3]
    rx2 = r[:, 3:4]
    h = ry2 - ry1
    w = rx2 - rx1
    cy = ry1 + 0.5 * h + d[0] * h
    cx = rx1 + 0.5 * w + d[1] * w
    h = h * jnp.exp(d[2])
    w = w * jnp.exp(d[3])
    y1 = cy - 0.5 * h
    x1 = cx - 0.5 * w
    y2 = y1 + h
    x2 = x1 + w

    wy1 = win_ref[0, 0]
    wx1 = win_ref[0, 1]
    wy2 = win_ref[0, 2]
    wx2 = win_ref[0, 3]
    y1 = jnp.clip(y1, wy1, wy2)
    x1 = jnp.clip(x1, wx1, wx2)
    y2 = jnp.clip(y2, wy1, wy2)
    x2 = jnp.clip(x2, wx1, wx2)

    keep = (idx > 0) & (m >= MIN_CONF)
    sc_ref[...] = jnp.where(keep, m, -1.0)
    y1_ref[...] = y1
    x1_ref[...] = x1
    y2_ref[...] = y2
    x2_ref[...] = x2
    cf_ref[...] = idx.astype(jnp.float32)


def _nms_body(y1_ref, x1_ref, y2_ref, x2_ref, cf_ref, sc_ref,
              oy1_ref, ox1_ref, oy2_ref, ox2_ref, ocl_ref, osc_ref):
    cf = cf_ref[...]
    ny1 = y1_ref[...] + 4.0 * cf
    nx1 = x1_ref[...] + 4.0 * cf
    ny2 = y2_ref[...] + 4.0 * cf
    nx2 = x2_ref[...] + 4.0 * cf
    area = (ny2 - ny1) * (nx2 - nx1)
    fio = (jax.lax.broadcasted_iota(jnp.int32, (ROWS, LANES), 0) * LANES
           + jax.lax.broadcasted_iota(jnp.int32, (ROWS, LANES), 1))
    orow = jax.lax.broadcasted_iota(jnp.int32, (8, LANES), 0)
    olane = jax.lax.broadcasted_iota(jnp.int32, (8, LANES), 1)

    def body(i, carry):
        sc, oy1, ox1, oy2, ox2, ocl, osc = carry
        m = jnp.max(sc)
        bi = jnp.min(jnp.where(sc == m, fio, NPAD))
        isbest = fio == bi
        by1 = jnp.sum(jnp.where(isbest, ny1, 0.0))
        bx1 = jnp.sum(jnp.where(isbest, nx1, 0.0))
        by2 = jnp.sum(jnp.where(isbest, ny2, 0.0))
        bx2 = jnp.sum(jnp.where(isbest, nx2, 0.0))
        bcl = jnp.sum(jnp.where(isbest, cf, 0.0))

        yy1 = jnp.maximum(by1, ny1)
        xx1 = jnp.maximum(bx1, nx1)
        yy2 = jnp.minimum(by2, ny2)
        xx2 = jnp.minimum(bx2, nx2)
        inter = jnp.maximum(yy2 - yy1, 0.0) * jnp.maximum(xx2 - xx1, 0.0)
        area_a = (by2 - by1) * (bx2 - bx1)
        iou = inter / (area_a + area - inter + 1e-8)
        sc = jnp.where((iou > NMS_THRESH) | isbest, -1.0, sc)

        validf = (m > 0.0).astype(jnp.float32)
        slot = (orow == 0) & (olane == i)
        oy1 = jnp.where(slot, (by1 - 4.0 * bcl) * validf, oy1)
        ox1 = jnp.where(slot, (bx1 - 4.0 * bcl) * validf, ox1)
        oy2 = jnp.where(slot, (by2 - 4.0 * bcl) * validf, oy2)
        ox2 = jnp.where(slot, (bx2 - 4.0 * bcl) * validf, ox2)
        ocl = jnp.where(slot, bcl * validf, ocl)
        osc = jnp.where(slot, m * validf, osc)
        return sc, oy1, ox1, oy2, ox2, ocl, osc

    z = jnp.zeros((8, LANES), jnp.float32)
    carry = (sc_ref[...], z, z, z, z, z, z)
    carry = jax.lax.fori_loop(0, MAXDET, body, carry)
    _, oy1, ox1, oy2, ox2, ocl, osc = carry
    oy1_ref[...] = oy1
    ox1_ref[...] = ox1
    oy2_ref[...] = oy2
    ox2_ref[...] = ox2
    ocl_ref[...] = ocl
    osc_ref[...] = osc


@jax.jit
def kernel(rois, probs, deltas, window):
    ntiles = N // TILE
    deltas_f = deltas.reshape(N, 4 * C)
    win = jnp.broadcast_to(window.reshape(1, 4), (8, 4))

    col = jax.ShapeDtypeStruct((N, 1), jnp.float32)
    row_spec = pl.BlockSpec((TILE, 1), lambda t: (t, 0))
    y1, x1, y2, x2, cf, sc = pl.pallas_call(
        _stage_a_body,
        grid=(ntiles,),
        in_specs=[
            pl.BlockSpec((TILE, 4), lambda t: (t, 0)),
            pl.BlockSpec((TILE, C), lambda t: (t, 0)),
            pl.BlockSpec((TILE, 4 * C), lambda t: (t, 0)),
            pl.BlockSpec((8, 4), lambda t: (0, 0)),
        ],
        out_specs=[row_spec] * 6,
        out_shape=[col] * 6,
    )(rois, probs, deltas_f, win)

    def relay(a, fill):
        a = jnp.concatenate(
            [a[:, 0], jnp.full((NPAD - N,), fill, jnp.float32)])
        return a.reshape(ROWS, LANES)

    y1p = relay(y1, 0.0)
    x1p = relay(x1, 0.0)
    y2p = relay(y2, 0.0)
    x2p = relay(x2, 0.0)
    cfp = relay(cf, 0.0)
    scp = relay(sc, -1.0)

    full = pl.BlockSpec((ROWS, LANES), lambda: (0, 0))
    osmall = jax.ShapeDtypeStruct((8, LANES), jnp.float32)
    ospec = pl.BlockSpec((8, LANES), lambda: (0, 0))
    oy1, ox1, oy2, ox2, ocl, osc = pl.pallas_call(
        _nms_body,
        in_specs=[full] * 6,
        out_specs=[ospec] * 6,
        out_shape=[osmall] * 6,
    )(y1p, x1p, y2p, x2p, cfp, scp)

    boxes = jnp.stack([oy1[0, :MAXDET], ox1[0, :MAXDET],
                       oy2[0, :MAXDET], ox2[0, :MAXDET]], axis=1)
    return jnp.concatenate(
        [boxes, ocl[0, :MAXDET, None], osc[0, :MAXDET, None]], axis=1)
